# baseline (speedup 1.0000x reference)
import jax
import jax.numpy as jnp
from jax import lax
from jax.experimental import pallas as pl
from jax.experimental.pallas import tpu as pltpu
from jax.experimental.pallas import tpu_sc as plsc

L = 200
D = 64
B = 4096
NC, NS = 2, 16
NW = NC * NS
UT = 128
UPL = B // UT
N_UNITS = L * UPL
UPW = N_UNITS // NW
NBUF = 2

def _emb_body(x_hbm, wt_hbm, pos_hbm, out_hbm, idx_v, rows_v, out_v, pos_s,
              sg0, sg1, ss0, ss1, si0, si1):
    sg = (sg0, sg1)
    ss = (ss0, ss1)
    si = (si0, si1)
    wid = lax.axis_index("s") * NC + lax.axis_index("c")
    base = wid * UPW

    def fire_idx(u, bb):
        l = u // UPL
        col = (u % UPL) * UT
        pltpu.async_copy(x_hbm.at[l, pl.ds(col, UT)], idx_v.at[bb], si[bb])
        pltpu.async_copy(pos_hbm.at[l], pos_s.at[bb], si[bb])

    def wait_idx(bb):
        pltpu.make_async_copy(x_hbm.at[0, pl.ds(0, UT)], idx_v.at[bb], si[bb]).wait()
        pltpu.make_async_copy(pos_hbm.at[0], pos_s.at[bb], si[bb]).wait()

    def fire_gathers(bb):
        pltpu.async_copy(wt_hbm.at[idx_v.at[bb]], rows_v.at[bb], sg[bb])

    def wait_gathers(bb):
        pltpu.make_async_copy(wt_hbm.at[pl.ds(0, UT)], rows_v.at[bb], sg[bb]).wait()

    def fire_stores(u, bb):
        l = u // UPL
        bt = u % UPL
        for dt in range(8):
            pltpu.async_copy(
                out_v.at[bb, pl.ds(dt * 1024, 1024)],
                out_hbm.at[l, dt, bt],
                ss[bb],
            )

    def wait_stores(bb):
        for dt in range(8):
            pltpu.make_async_copy(
                out_v.at[bb, pl.ds(dt * 1024, 1024)], out_hbm.at[0, dt, 0], ss[bb]
            ).wait()

    fire_idx(base, 0)
    wait_idx(0)
    fire_gathers(0)
    fire_idx(base + 1, 1)

    def outer(ci, carry):
        for b in range(NBUF):
            i = ci * NBUF + b
            u = base + i
            bp = (b + 1) % NBUF

            if b < NBUF - 1:
                wait_idx(bp)

                @pl.when(ci >= 1)
                def _():
                    wait_stores(bp)

                fire_gathers(bp)
            else:
                @pl.when(ci < UPW // NBUF - 1)
                def _():
                    wait_idx(bp)
                    wait_stores(bp)
                    fire_gathers(bp)

            wait_gathers(b)
            iota = lax.iota(jnp.int32, 16)
            for db in range(D // 16):
                dcol = [lax.rem(iota + k, 16) + db * 16 for k in range(16)]
                posd = [
                    plsc.load_gather(pos_s.at[b], [dcol[k]]) for k in range(16)
                ]
                oaddrc = [
                    (dcol[k] // 8) * 1024 + lax.rem(dcol[k], 8) * 128 + iota
                    for k in range(16)
                ]
                _b = b

                @plsc.parallel_loop(0, UT // 16, 1, unroll=4)
                def btg_body(btg, _b=_b, _dcol=dcol, _posd=posd, _oaddrc=oaddrc):
                    tok = lax.iota(jnp.int32, 16) + btg * 16
                    lane0 = btg * 16
                    for k in range(16):
                        v = plsc.load_gather(rows_v.at[_b], [tok, _dcol[k]])
                        plsc.store_scatter(
                            out_v.at[_b], [_oaddrc[k] + lane0], v + _posd[k]
                        )

            fire_stores(u, b)

            @pl.when(ci < UPW // NBUF - 1)
            def _():
                fire_idx(u + 2, b)
        return carry

    lax.fori_loop(0, UPW // NBUF, outer, 0)
    for b in range(NBUF):
        wait_stores(b)


_OFF = 487936
_T = 512512


def _table_transpose_body(a_ref, b_ref, o_ref):
    o_ref[...] = jnp.concatenate([a_ref[...].T, b_ref[...].T], axis=1)


def _prep_table(word_table):
    wtt = jnp.swapaxes(word_table, 0, 1)
    br = 512
    nb = _T // br
    wtc = pl.pallas_call(
        _table_transpose_body,
        grid=(nb,),
        in_specs=[
            pl.BlockSpec((D, br), lambda r: (0, r)),
            pl.BlockSpec((D, br), lambda r: (0, _OFF // 512 + r)),
        ],
        out_specs=pl.BlockSpec((br, 2 * D), lambda r: (r, 0)),
        out_shape=jax.ShapeDtypeStruct((_T, 2 * D), jnp.float32),
    )(wtt, wtt)
    return wtc.reshape(2 * _T, D)


def kernel(x, word_table, pos_table):
    xt = jnp.swapaxes(x, 0, 1).astype(jnp.int32)
    xt = 2 * xt - jnp.where(xt >= _T, 2 * _OFF - 1, 0)
    wt = _prep_table(word_table)
    mesh = plsc.VectorSubcoreMesh(core_axis_name="c", subcore_axis_name="s")
    out = pl.kernel(
        _emb_body,
        out_type=jax.ShapeDtypeStruct((L, 8, B // 128, 1024), jnp.float32),
        mesh=mesh,
        compiler_params=pltpu.CompilerParams(
            use_tc_tiling_on_sc=False, needs_layout_passes=False
        ),
        scratch_types=[
            pltpu.VMEM((NBUF, UT), jnp.int32),
            pltpu.VMEM((NBUF, UT, D), jnp.float32),
            pltpu.VMEM((NBUF, 8 * 1024), jnp.float32),
            pltpu.VMEM((NBUF, D), jnp.float32),
        ] + [pltpu.SemaphoreType.DMA] * 6,
    )(xt, wt, pos_table.astype(jnp.float32))
    out5 = out.reshape(L, 8, B // 128, 8, 128)
    return out5.transpose(2, 4, 0, 1, 3).reshape(B, L, D)


# speedup vs baseline: 1.7680x; 1.7680x over previous
import jax
import jax.numpy as jnp
from jax import lax
from jax.experimental import pallas as pl
from jax.experimental.pallas import tpu as pltpu
from jax.experimental.pallas import tpu_sc as plsc

L = 200
D = 64
B = 4096
NC, NS = 2, 16
NW = NC * NS
UT = 128
UPL = B // UT
N_UNITS = L * UPL
UPW = N_UNITS // NW
NBUF = 2

def _emb_body(x_hbm, wt_hbm, pos_hbm, out_hbm, idx_v, rows_v, out_v, pos_s,
              sg0, sg1, ss0, ss1, si0, si1):
    sg = (sg0, sg1)
    ss = (ss0, ss1)
    si = (si0, si1)
    wid = lax.axis_index("s") * NC + lax.axis_index("c")
    base = wid * UPW

    def fire_idx(u, bb):
        l = u // UPL
        col = (u % UPL) * UT
        pltpu.async_copy(x_hbm.at[l, pl.ds(col, UT)], idx_v.at[bb], si[bb])
        pltpu.async_copy(pos_hbm.at[l], pos_s.at[bb], si[bb])

    def wait_idx(bb):
        pltpu.make_async_copy(x_hbm.at[0, pl.ds(0, UT)], idx_v.at[bb], si[bb]).wait()
        pltpu.make_async_copy(pos_hbm.at[0], pos_s.at[bb], si[bb]).wait()

    def fire_gathers(bb):
        pltpu.async_copy(wt_hbm.at[idx_v.at[bb]], rows_v.at[bb], sg[bb])

    def wait_gathers(bb):
        pltpu.make_async_copy(wt_hbm.at[pl.ds(0, UT)], rows_v.at[bb], sg[bb]).wait()

    def fire_stores(u, bb):
        l = u // UPL
        bt = u % UPL
        for dt in range(8):
            pltpu.async_copy(
                out_v.at[bb, pl.ds(dt * 1024, 1024)],
                out_hbm.at[l, dt, bt],
                ss[bb],
            )

    def wait_stores(bb):
        for dt in range(8):
            pltpu.make_async_copy(
                out_v.at[bb, pl.ds(dt * 1024, 1024)], out_hbm.at[0, dt, 0], ss[bb]
            ).wait()

    fire_idx(base, 0)
    wait_idx(0)
    fire_gathers(0)
    fire_idx(base + 1, 1)

    def outer(ci, carry):
        for b in range(NBUF):
            i = ci * NBUF + b
            u = base + i
            bp = (b + 1) % NBUF

            if b < NBUF - 1:
                wait_idx(bp)

                @pl.when(ci >= 1)
                def _():
                    wait_stores(bp)

                fire_gathers(bp)
            else:
                @pl.when(ci < UPW // NBUF - 1)
                def _():
                    wait_idx(bp)
                    wait_stores(bp)
                    fire_gathers(bp)

            wait_gathers(b)
            iota = lax.iota(jnp.int32, 16)
            for db in range(D // 16):
                dcol = [lax.rem(iota + k, 16) + db * 16 for k in range(16)]
                posd = [
                    plsc.load_gather(pos_s.at[b], [dcol[k]]) for k in range(16)
                ]
                oaddrc = [
                    (dcol[k] // 8) * 1024 + lax.rem(dcol[k], 8) * 128 + iota
                    for k in range(16)
                ]
                _b = b

                @plsc.parallel_loop(0, UT // 16, 1, unroll=4)
                def btg_body(btg, _b=_b, _dcol=dcol, _posd=posd, _oaddrc=oaddrc):
                    tok = lax.iota(jnp.int32, 16) + btg * 16
                    lane0 = btg * 16
                    for k in range(16):
                        v = plsc.load_gather(rows_v.at[_b], [tok, _dcol[k]])
                        plsc.store_scatter(
                            out_v.at[_b], [_oaddrc[k] + lane0], v + _posd[k]
                        )

            fire_stores(u, b)

            @pl.when(ci < UPW // NBUF - 1)
            def _():
                fire_idx(u + 2, b)
        return carry

    lax.fori_loop(0, UPW // NBUF, outer, 0)
    for b in range(NBUF):
        wait_stores(b)


_OFF = 491520
_T = 516096


def _table_transpose_body(a_ref, b_ref, o_ref):
    o_ref[...] = jnp.concatenate([a_ref[...].T, b_ref[...].T], axis=1)


def _prep_table(word_table):
    wtt = jnp.swapaxes(word_table, 0, 1)
    br = 8192
    nb = _T // br
    wtc = pl.pallas_call(
        _table_transpose_body,
        grid=(nb,),
        in_specs=[
            pl.BlockSpec((D, br), lambda r: (0, r)),
            pl.BlockSpec((D, br), lambda r: (0, _OFF // 8192 + r)),
        ],
        out_specs=pl.BlockSpec((br, 2 * D), lambda r: (r, 0)),
        out_shape=jax.ShapeDtypeStruct((_T, 2 * D), jnp.float32),
    )(wtt, wtt)
    return wtc.reshape(2 * _T, D)


def kernel(x, word_table, pos_table):
    xt = jnp.swapaxes(x, 0, 1).astype(jnp.int32)
    xt = 2 * xt - jnp.where(xt >= _T, 2 * _OFF - 1, 0)
    wt = _prep_table(word_table)
    mesh = plsc.VectorSubcoreMesh(core_axis_name="c", subcore_axis_name="s")
    out = pl.kernel(
        _emb_body,
        out_type=jax.ShapeDtypeStruct((L, 8, B // 128, 1024), jnp.float32),
        mesh=mesh,
        compiler_params=pltpu.CompilerParams(
            use_tc_tiling_on_sc=False, needs_layout_passes=False
        ),
        scratch_types=[
            pltpu.VMEM((NBUF, UT), jnp.int32),
            pltpu.VMEM((NBUF, UT, D), jnp.float32),
            pltpu.VMEM((NBUF, 8 * 1024), jnp.float32),
            pltpu.VMEM((NBUF, D), jnp.float32),
        ] + [pltpu.SemaphoreType.DMA] * 6,
    )(xt, wt, pos_table.astype(jnp.float32))
    out5 = out.reshape(L, 8, B // 128, 8, 128)
    return out5.transpose(2, 4, 0, 1, 3).reshape(B, L, D)
